# baseline (speedup 1.0000x reference)
import jax
import jax.numpy as jnp
from jax.experimental import pallas as pl
from jax.experimental.pallas import tpu as pltpu

_BLOCK_B = 4096


def _mrrd_block(r_ref, out_ref):
    r = r_ref[...]
    neg = r < 0
    cnt = jnp.sum(neg.astype(jnp.int32), axis=1, keepdims=True)
    even = (cnt & 1) == 0
    out_ref[...] = jnp.where(jnp.logical_and(neg, even), 1.0, 0.0).astype(
        out_ref.dtype)


def kernel(r, PermGrp):
    del PermGrp
    b, n = r.shape
    block_b = _BLOCK_B if b % _BLOCK_B == 0 else b
    return pl.pallas_call(
        _mrrd_block,
        grid=(b // block_b,),
        in_specs=[pl.BlockSpec((block_b, n), lambda i: (i, 0))],
        out_specs=pl.BlockSpec((block_b, n), lambda i: (i, 0)),
        out_shape=jax.ShapeDtypeStruct((b, n), r.dtype),
        compiler_params=pltpu.CompilerParams(
            dimension_semantics=("parallel",)),
    )(r)


# speedup vs baseline: 1.3024x; 1.3024x over previous
import jax
import jax.numpy as jnp
from jax.experimental import pallas as pl
from jax.experimental.pallas import tpu as pltpu

_BLOCK_B = 2048


def _mrrd_block(r_ref, out_ref):
    r = r_ref[...]
    neg = r < 0
    cnt = jnp.sum(neg.astype(jnp.int32), axis=1, keepdims=True)
    even = (cnt & 1) == 0
    out_ref[...] = jnp.where(jnp.logical_and(neg, even), 1.0, 0.0).astype(
        out_ref.dtype)


def kernel(r, PermGrp):
    del PermGrp
    b, n = r.shape
    block_b = _BLOCK_B if b % _BLOCK_B == 0 else b
    return pl.pallas_call(
        _mrrd_block,
        grid=(b // block_b,),
        in_specs=[pl.BlockSpec((block_b, n), lambda i: (i, 0))],
        out_specs=pl.BlockSpec((block_b, n), lambda i: (i, 0)),
        out_shape=jax.ShapeDtypeStruct((b, n), r.dtype),
        compiler_params=pltpu.CompilerParams(
            dimension_semantics=("parallel",)),
    )(r)
